# Initial kernel scaffold; baseline (speedup 1.0000x reference)
import jax
import jax.numpy as jnp
from jax.experimental import pallas as pl


def kernel(input_ids, W, b, pos_table, gamma, beta):
    raise NotImplementedError("write your pallas kernel here")


# speedup vs baseline: 4.8168x; 4.8168x over previous
import jax
import jax.numpy as jnp
from jax.experimental import pallas as pl

_EPS = 1e-12
_TS = 512


def _body(x_ref, w_ref, b_ref, pos_ref, g_ref, beta_ref, o_ref):
    x = x_ref[0]
    y = jnp.dot(x, w_ref[...], preferred_element_type=jnp.float32)
    y = y + b_ref[...] + pos_ref[...]
    mean = jnp.mean(y, axis=-1, keepdims=True)
    yc = y - mean
    var = jnp.mean(yc * yc, axis=-1, keepdims=True)
    inv = jax.lax.rsqrt(var + _EPS)
    o_ref[0] = (yc * inv) * g_ref[...] + beta_ref[...]


@jax.jit
def kernel(input_ids, W, b, pos_table, gamma, beta):
    B, S, D = input_ids.shape
    H = W.shape[1]
    ts = min(_TS, S)
    grid = (S // ts, B)

    b2 = b.reshape(1, H)
    g2 = gamma.reshape(1, H)
    beta2 = beta.reshape(1, H)
    pos = pos_table[:S]

    return pl.pallas_call(
        _body,
        grid=grid,
        in_specs=[
            pl.BlockSpec((1, ts, D), lambda j, i: (i, j, 0)),
            pl.BlockSpec((D, H), lambda j, i: (0, 0)),
            pl.BlockSpec((1, H), lambda j, i: (0, 0)),
            pl.BlockSpec((ts, H), lambda j, i: (j, 0)),
            pl.BlockSpec((1, H), lambda j, i: (0, 0)),
            pl.BlockSpec((1, H), lambda j, i: (0, 0)),
        ],
        out_specs=pl.BlockSpec((1, ts, H), lambda j, i: (i, j, 0)),
        out_shape=jax.ShapeDtypeStruct((B, S, H), jnp.float32),
    )(input_ids, W, b2, pos, g2, beta2)


# speedup vs baseline: 4.8402x; 1.0049x over previous
import jax
import jax.numpy as jnp
from jax.experimental import pallas as pl
from jax.experimental.pallas import tpu as pltpu

_EPS = 1e-12
_TS = 512


def _body(x_ref, w_ref, b_ref, pos_ref, g_ref, beta_ref, o_ref):
    x = x_ref[0]
    y = jnp.dot(x, w_ref[...], preferred_element_type=jnp.float32)
    y = y + b_ref[...] + pos_ref[...]
    mean = jnp.mean(y, axis=-1, keepdims=True)
    yc = y - mean
    var = jnp.mean(yc * yc, axis=-1, keepdims=True)
    inv = jax.lax.rsqrt(var + _EPS)
    o_ref[0] = (yc * inv) * g_ref[...] + beta_ref[...]


@jax.jit
def kernel(input_ids, W, b, pos_table, gamma, beta):
    B, S, D = input_ids.shape
    H = W.shape[1]
    ts = min(_TS, S)
    grid = (S // ts, B)

    b2 = b.reshape(1, H)
    g2 = gamma.reshape(1, H)
    beta2 = beta.reshape(1, H)
    pos = pos_table[:S]

    return pl.pallas_call(
        _body,
        grid=grid,
        in_specs=[
            pl.BlockSpec((1, ts, D), lambda j, i: (i, j, 0)),
            pl.BlockSpec((D, H), lambda j, i: (0, 0)),
            pl.BlockSpec((1, H), lambda j, i: (0, 0)),
            pl.BlockSpec((ts, H), lambda j, i: (j, 0)),
            pl.BlockSpec((1, H), lambda j, i: (0, 0)),
            pl.BlockSpec((1, H), lambda j, i: (0, 0)),
        ],
        out_specs=pl.BlockSpec((1, ts, H), lambda j, i: (i, j, 0)),
        out_shape=jax.ShapeDtypeStruct((B, S, H), jnp.float32),
        compiler_params=pltpu.CompilerParams(
            dimension_semantics=("parallel", "parallel"),
        ),
    )(input_ids, W, b2, pos, g2, beta2)
